# baseline (speedup 1.0000x reference)
import functools

import jax
import jax.numpy as jnp
from jax import lax
from jax.experimental import pallas as pl
from jax.experimental.pallas import tpu as pltpu
from jax.experimental.pallas import tpu_sc as plsc


def _lane_iota():
    return lax.broadcasted_iota(jnp.int32, (16,), 0)


def _make_sc_src_gather(t, d, chunk):
    info = plsc.get_sparse_core_info()
    nc, ns = info.num_cores, info.num_subcores
    nw = nc * ns
    tpw = t // nw
    assert t % nw == 0 and tpw % chunk == 0 and chunk % 16 == 0
    n_chunks = tpw // chunk

    mesh = plsc.VectorSubcoreMesh(core_axis_name="c", subcore_axis_name="s")

    @functools.partial(
        pl.kernel,
        mesh=mesh,
        compiler_params=pltpu.CompilerParams(
            use_tc_tiling_on_sc=True, needs_layout_passes=False
        ),
        out_type=jax.ShapeDtypeStruct((t, d), jnp.float32),
        scratch_types=[
            pltpu.VMEM((tpw,), jnp.int32),
            pltpu.VMEM((tpw + chunk,), jnp.int32),
            pltpu.VMEM((2, chunk, 2 * d), jnp.float32),
            pltpu.VMEM((chunk, d), jnp.float32),
            pltpu.SemaphoreType.DMA,
            pltpu.SemaphoreType.DMA,
        ],
    )
    def src_gather(table2, idx_hbm, out_hbm, idx_v, ti_v, rows_v, ob_v, sem0, sem1):
        wid = lax.axis_index("s") * nc + lax.axis_index("c")
        tok0 = wid * tpw
        pltpu.sync_copy(idx_hbm.at[pl.ds(tok0, tpw)], idx_v)
        for g in range(tpw // 16):
            v = idx_v[pl.ds(g * 16, 16)]
            ti_v[pl.ds(g * 16, 16)] = lax.shift_right_logical(v, 1)
        zeros16 = jnp.zeros((16,), jnp.int32)
        for g in range(chunk // 16):
            ti_v[pl.ds(tpw + g * 16, 16)] = zeros16

        sems = (sem0, sem1)

        def fire(j, buf):
            off = pl.multiple_of(j * chunk, chunk)
            pltpu.async_copy(
                table2.at[ti_v.at[pl.ds(off, chunk)]], rows_v.at[buf], sems[buf]
            )

        def drain(buf):
            pltpu.make_async_copy(
                table2.at[ti_v.at[pl.ds(0, chunk)]], rows_v.at[buf], sems[buf]
            ).wait()

        def extract_and_store(j, buf):
            for g in range(chunk // 16):
                off = pl.multiple_of(j * chunk + g * 16, 16)
                tok16 = _lane_iota() + g * 16
                half16 = lax.bitwise_and(idx_v[pl.ds(off, 16)], 1) * d
                for c in range(d):
                    vals = plsc.load_gather(rows_v.at[buf], [tok16, half16 + c])
                    plsc.store_scatter(
                        ob_v, [tok16, jnp.full((16,), c, jnp.int32)], vals
                    )
            out_off = pl.multiple_of(tok0 + j * chunk, chunk)
            pltpu.sync_copy(ob_v, out_hbm.at[pl.ds(out_off, chunk)])

        fire(0, 0)

        def pair_body(jj, carry):
            for b2 in range(2):
                j = jj * 2 + b2
                fire(j + 1, 1 - b2)
                drain(b2)
                extract_and_store(j, b2)
            return carry

        assert n_chunks % 2 == 0
        lax.fori_loop(0, n_chunks // 2, pair_body, 0)
        drain(0)

    return src_gather


def _make_tc_logits_t(b, s, v, d, n):
    def body(idx_ref, tgt_t_ref, log_t_ref, te_ref, ol_ref):
        idx_row = idx_ref[0]
        oh = (lax.broadcasted_iota(jnp.int32, (v, b), 0) == idx_row).astype(
            jnp.bfloat16
        )
        te = jnp.dot(tgt_t_ref[...], oh, preferred_element_type=jnp.float32)
        te_ref[0] = te
        ol_ref[0] = jnp.dot(
            log_t_ref[...], te.astype(jnp.bfloat16),
            preferred_element_type=jnp.float32,
        )

    return pl.pallas_call(
        body,
        grid=(s,),
        in_specs=[
            pl.BlockSpec((1, 1, b), lambda i: (i, 0, 0)),
            pl.BlockSpec((d, v), lambda i: (0, 0)),
            pl.BlockSpec((n, d), lambda i: (0, 0)),
        ],
        out_specs=[
            pl.BlockSpec((1, d, b), lambda i: (i, 0, 0)),
            pl.BlockSpec((1, n, b), lambda i: (i, 0, 0)),
        ],
        out_shape=[
            jax.ShapeDtypeStruct((s, d, b), jnp.float32),
            jax.ShapeDtypeStruct((s, n, b), jnp.float32),
        ],
    )


def kernel(source_enumerate, target_enumerate, src_embs, tgt_embs, logits):
    b, s = source_enumerate.shape
    t = b * s
    src_v, d = src_embs.shape
    tgt_v = tgt_embs.shape[0]
    n = logits.shape[1]

    src_idx = source_enumerate.reshape(t).astype(jnp.int32)
    table2 = jnp.concatenate([src_embs[0::2], src_embs[1::2]], axis=1)
    src2d = _make_sc_src_gather(t, d, chunk=160)(table2, src_idx)

    idx_t = target_enumerate.astype(jnp.int32).T.reshape(s, 1, b)
    tgt_t = tgt_embs.T.astype(jnp.bfloat16)
    log_t = logits.T.astype(jnp.bfloat16)
    te_t, ol_t = _make_tc_logits_t(b, s, tgt_v, d, n)(idx_t, tgt_t, log_t)

    return (
        src2d.reshape(b, s, d),
        jnp.transpose(te_t, (2, 0, 1)),
        jnp.transpose(ol_t, (2, 0, 1)),
    )


# speedup vs baseline: 1.7739x; 1.7739x over previous
import functools

import jax
import jax.numpy as jnp
from jax import lax
from jax.experimental import pallas as pl
from jax.experimental.pallas import tpu as pltpu
from jax.experimental.pallas import tpu_sc as plsc


def _lane_iota():
    return lax.broadcasted_iota(jnp.int32, (16,), 0)


def _make_sc_src_gather_t(b, s, d, v):
    info = plsc.get_sparse_core_info()
    nc, ns = info.num_cores, info.num_subcores
    nw = nc * ns
    cpw = d // nw
    assert d % nw == 0 and b % 16 == 0

    mesh = plsc.VectorSubcoreMesh(core_axis_name="c", subcore_axis_name="s")

    @functools.partial(
        pl.kernel,
        mesh=mesh,
        compiler_params=pltpu.CompilerParams(
            use_tc_tiling_on_sc=True, needs_layout_passes=False
        ),
        out_type=jax.ShapeDtypeStruct((s, d, b), jnp.float32),
        scratch_types=[
            pltpu.VMEM((s, b), jnp.int32),
            pltpu.VMEM((b,), jnp.int32),
            pltpu.VMEM((b,), jnp.int32),
            pltpu.VMEM((b,), jnp.float32),
            pltpu.VMEM((b,), jnp.float32),
            pltpu.SemaphoreType.DMA,
            pltpu.SemaphoreType.DMA,
        ],
    )
    def src_gather(
        table1d, idxt_hbm, out_hbm, idx_v, ic0, ic1, vals0, vals1, sem0, sem1
    ):
        wid = lax.axis_index("s") * nc + lax.axis_index("c")
        c0 = wid * cpw
        pltpu.sync_copy(idxt_hbm, idx_v)
        sems = (sem0, sem1)
        ics = (ic0, ic1)
        vals = (vals0, vals1)
        n_tasks = cpw * s

        def task_sc(p):
            return p // cpw, c0 + lax.rem(p, cpw)

        def prep_and_fire(p, buf):
            sp, cp = task_sc(p)
            base = cp * v
            for g in range(b // 16):
                ics[buf][pl.ds(g * 16, 16)] = idx_v[sp, pl.ds(g * 16, 16)] + base
            pltpu.async_copy(table1d.at[ics[buf]], vals[buf], sems[buf])

        def drain(buf):
            pltpu.make_async_copy(
                table1d.at[ics[buf]], vals[buf], sems[buf]
            ).wait()

        def store(p, buf):
            sp, cp = task_sc(p)
            pltpu.sync_copy(vals[buf], out_hbm.at[sp, cp])

        prep_and_fire(0, 0)

        def pair_body(jj, carry):
            for b2 in range(2):
                p = jj * 2 + b2
                nxt = lax.min(p + 1, n_tasks - 1)
                prep_and_fire(nxt, 1 - b2)
                drain(b2)
                store(p, b2)
            return carry

        assert n_tasks % 2 == 0
        lax.fori_loop(0, n_tasks // 2, pair_body, 0)
        drain(0)

    return src_gather


def _make_tc_logits_t(b, s, v, d, n):
    def body(idx_ref, tgt_t_ref, log_t_ref, te_ref, ol_ref):
        idx_row = idx_ref[0]
        oh = (lax.broadcasted_iota(jnp.int32, (v, b), 0) == idx_row).astype(
            jnp.bfloat16
        )
        te = jnp.dot(tgt_t_ref[...], oh, preferred_element_type=jnp.float32)
        te_ref[0] = te
        ol_ref[0] = jnp.dot(
            log_t_ref[...], te.astype(jnp.bfloat16),
            preferred_element_type=jnp.float32,
        )

    return pl.pallas_call(
        body,
        grid=(s,),
        in_specs=[
            pl.BlockSpec((1, 1, b), lambda i: (i, 0, 0)),
            pl.BlockSpec((d, v), lambda i: (0, 0)),
            pl.BlockSpec((n, d), lambda i: (0, 0)),
        ],
        out_specs=[
            pl.BlockSpec((1, d, b), lambda i: (i, 0, 0)),
            pl.BlockSpec((1, n, b), lambda i: (i, 0, 0)),
        ],
        out_shape=[
            jax.ShapeDtypeStruct((s, d, b), jnp.float32),
            jax.ShapeDtypeStruct((s, n, b), jnp.float32),
        ],
    )


def kernel(source_enumerate, target_enumerate, src_embs, tgt_embs, logits):
    b, s = source_enumerate.shape
    t = b * s
    src_v, d = src_embs.shape
    tgt_v = tgt_embs.shape[0]
    n = logits.shape[1]

    table1d = src_embs.T.reshape(src_v * d)
    src_idx_t = source_enumerate.astype(jnp.int32).T
    src_t = _make_sc_src_gather_t(b, s, d, src_v)(table1d, src_idx_t)

    idx_t = target_enumerate.astype(jnp.int32).T.reshape(s, 1, b)
    tgt_t = tgt_embs.T.astype(jnp.bfloat16)
    log_t = logits.T.astype(jnp.bfloat16)
    te_t, ol_t = _make_tc_logits_t(b, s, tgt_v, d, n)(idx_t, tgt_t, log_t)

    return (
        jnp.transpose(src_t, (2, 0, 1)),
        jnp.transpose(te_t, (2, 0, 1)),
        jnp.transpose(ol_t, (2, 0, 1)),
    )


# speedup vs baseline: 8.0323x; 4.5281x over previous
import functools

import jax
import jax.numpy as jnp
from jax import lax
from jax.experimental import pallas as pl
from jax.experimental.pallas import tpu as pltpu
from jax.experimental.pallas import tpu_sc as plsc


def _make_sc_gather(t, d):
    info = plsc.get_sparse_core_info()
    nw = info.num_cores * info.num_subcores
    tpw = t // nw
    chunk = 80
    assert t % nw == 0 and tpw % chunk == 0
    n_chunks = tpw // chunk

    mesh = plsc.VectorSubcoreMesh(core_axis_name="c", subcore_axis_name="s")

    @functools.partial(
        pl.kernel,
        mesh=mesh,
        compiler_params=pltpu.CompilerParams(use_tc_tiling_on_sc=False),
        out_type=jax.ShapeDtypeStruct((t, d), jnp.float32),
        scratch_types=[
            pltpu.VMEM((tpw,), jnp.int32),
            pltpu.VMEM((tpw, d), jnp.float32),
            pltpu.SemaphoreType.DMA,
        ],
    )
    def gather(table_hbm, idx_hbm, out_hbm, idx_v, rows_v, sem):
        wid = lax.axis_index("s") * info.num_cores + lax.axis_index("c")
        base = wid * tpw
        pltpu.sync_copy(idx_hbm.at[pl.ds(base, tpw)], idx_v)
        for j in range(n_chunks):
            pltpu.async_copy(
                table_hbm.at[idx_v.at[pl.ds(j * chunk, chunk)]],
                rows_v.at[pl.ds(j * chunk, chunk)],
                sem,
            )
        for j in range(n_chunks):
            pltpu.make_async_copy(
                table_hbm.at[idx_v.at[pl.ds(j * chunk, chunk)]],
                rows_v.at[pl.ds(j * chunk, chunk)],
                sem,
            ).wait()
        pltpu.sync_copy(rows_v, out_hbm.at[pl.ds(base, tpw)])

    return gather


def _make_tc_logits(t, v, d, n, blk=256):
    assert t % blk == 0

    def body(idx_ref, tgt_ref, log_ref, te_ref, ol_ref):
        idx = idx_ref[0, 0, :]
        oh = (
            lax.broadcasted_iota(jnp.int32, (blk, v), 1) == idx[:, None]
        ).astype(jnp.bfloat16)
        te = jnp.dot(oh, tgt_ref[...], preferred_element_type=jnp.float32)
        te_ref[...] = te
        ol_ref[...] = jnp.dot(
            te.astype(jnp.bfloat16), log_ref[...],
            preferred_element_type=jnp.float32,
        )

    return pl.pallas_call(
        body,
        grid=(t // blk,),
        in_specs=[
            pl.BlockSpec((1, 1, blk), lambda i: (i, 0, 0)),
            pl.BlockSpec((v, d), lambda i: (0, 0)),
            pl.BlockSpec((d, n), lambda i: (0, 0)),
        ],
        out_specs=[
            pl.BlockSpec((blk, d), lambda i: (i, 0)),
            pl.BlockSpec((blk, n), lambda i: (i, 0)),
        ],
        out_shape=[
            jax.ShapeDtypeStruct((t, d), jnp.float32),
            jax.ShapeDtypeStruct((t, n), jnp.float32),
        ],
    )


def kernel(source_enumerate, target_enumerate, src_embs, tgt_embs, logits):
    b, s = source_enumerate.shape
    t = b * s
    d = src_embs.shape[1]
    tgt_v = tgt_embs.shape[0]
    n = logits.shape[1]

    src_idx = source_enumerate.reshape(t).astype(jnp.int32)
    src_emb = _make_sc_gather(t, d)(src_embs, src_idx)

    blk = 256
    idx3 = target_enumerate.astype(jnp.int32).reshape(t // blk, 1, blk)
    tgt_bf = tgt_embs.astype(jnp.bfloat16)
    log_bf = logits.astype(jnp.bfloat16)
    te, ol = _make_tc_logits(t, tgt_v, d, n, blk)(idx3, tgt_bf, log_bf)

    return (
        src_emb.reshape(b, s, d),
        te.reshape(b, s, d),
        ol.reshape(b, s, n),
    )


# speedup vs baseline: 8.3844x; 1.0438x over previous
import functools

import jax
import jax.numpy as jnp
from jax import lax
from jax.experimental import pallas as pl
from jax.experimental.pallas import tpu as pltpu
from jax.experimental.pallas import tpu_sc as plsc


def _make_sc_pair_gather(t, dd):
    info = plsc.get_sparse_core_info()
    nw = info.num_cores * info.num_subcores
    tpw = t // nw
    chunk = 80
    assert t % nw == 0 and tpw % chunk == 0
    n_chunks = tpw // chunk

    mesh = plsc.VectorSubcoreMesh(core_axis_name="c", subcore_axis_name="s")

    @functools.partial(
        pl.kernel,
        mesh=mesh,
        out_type=jax.ShapeDtypeStruct((t, dd), jnp.float32),
        scratch_types=[
            pltpu.VMEM((tpw,), jnp.int32),
            pltpu.VMEM((chunk, dd), jnp.float32),
            pltpu.VMEM((chunk, dd), jnp.float32),
            pltpu.SemaphoreType.DMA,
            pltpu.SemaphoreType.DMA,
        ],
    )
    def gather(table_hbm, idx_hbm, out_hbm, idx_v, r0, r1, s0, s1):
        wid = lax.axis_index("s") * info.num_cores + lax.axis_index("c")
        base = wid * tpw
        pltpu.sync_copy(idx_hbm.at[pl.ds(base, tpw)], idx_v)
        rows = (r0, r1)
        sems = (s0, s1)

        def fire(j):
            pltpu.async_copy(
                table_hbm.at[idx_v.at[pl.ds(j * chunk, chunk)]],
                rows[j % 2],
                sems[j % 2],
            )

        fire(0)
        for j in range(n_chunks):
            if j + 1 < n_chunks:
                fire(j + 1)
            pltpu.make_async_copy(
                table_hbm.at[idx_v.at[pl.ds(j * chunk, chunk)]],
                rows[j % 2],
                sems[j % 2],
            ).wait()
            pltpu.sync_copy(
                rows[j % 2], out_hbm.at[pl.ds(base + j * chunk, chunk)]
            )

    return gather


def _make_tc_logits(b, s, v, d, n, bb=16):
    assert b % bb == 0

    def body(idx_ref, tgt_ref, log_ref, te_ref, ol_ref):
        idx = idx_ref[0]
        oh = (
            lax.broadcasted_iota(jnp.int32, (bb, s, v), 2) == idx
        ).astype(jnp.bfloat16)
        te = lax.dot_general(
            oh, tgt_ref[...], (((2,), (0,)), ((), ())),
            preferred_element_type=jnp.float32,
        )
        te_ref[...] = te
        ol_ref[...] = lax.dot_general(
            te.astype(jnp.bfloat16), log_ref[...], (((2,), (0,)), ((), ())),
            preferred_element_type=jnp.float32,
        )

    return pl.pallas_call(
        body,
        grid=(b // bb,),
        in_specs=[
            pl.BlockSpec((1, bb, s, 1), lambda i: (i, 0, 0, 0)),
            pl.BlockSpec((v, d), lambda i: (0, 0)),
            pl.BlockSpec((d, n), lambda i: (0, 0)),
        ],
        out_specs=[
            pl.BlockSpec((bb, s, d), lambda i: (i, 0, 0)),
            pl.BlockSpec((bb, s, n), lambda i: (i, 0, 0)),
        ],
        out_shape=[
            jax.ShapeDtypeStruct((b, s, d), jnp.float32),
            jax.ShapeDtypeStruct((b, s, n), jnp.float32),
        ],
    )


def kernel(source_enumerate, target_enumerate, src_embs, tgt_embs, logits):
    b, s = source_enumerate.shape
    t = b * s
    src_v, d = src_embs.shape
    tgt_v = tgt_embs.shape[0]
    n = logits.shape[1]

    src_idx = source_enumerate.reshape(t).astype(jnp.int32)
    table2 = src_embs.reshape(src_v // 2, 2 * d)
    pairs = _make_sc_pair_gather(t, 2 * d)(table2, src_idx >> 1)
    odd = (src_idx & 1)[:, None].astype(jnp.bool_)
    src_emb = jnp.where(odd, pairs[:, d:], pairs[:, :d]).reshape(b, s, d)

    bb = 16
    idx4 = target_enumerate.astype(jnp.int32).reshape(b // bb, bb, s, 1)
    tgt_bf = tgt_embs.astype(jnp.bfloat16)
    log_bf = logits.astype(jnp.bfloat16)
    te, ol = _make_tc_logits(b, s, tgt_v, d, n, bb)(idx4, tgt_bf, log_bf)

    return (src_emb, te, ol)
